# Initial kernel scaffold; baseline (speedup 1.0000x reference)
import jax
import jax.numpy as jnp
from jax.experimental import pallas as pl


def kernel(xyz, points, params):
    raise NotImplementedError("write your pallas kernel here")


# speedup vs baseline: 8.3737x; 8.3737x over previous
import functools

import jax
import jax.numpy as jnp
from jax.experimental import pallas as pl

B = 8
N = 512
S = 128
EPS = 1e-5
RADII = (0.2, 0.4, 0.8)
KS = (32, 64, 128)


def _fps_kernel(xyz_ref, newxyz_ref):
    x = xyz_ref[:, 0, :]
    y = xyz_ref[:, 1, :]
    z = xyz_ref[:, 2, :]
    lane = jax.lax.broadcasted_iota(jnp.int32, (B, N), 1)

    def body(i, state):
        distance, farthest = state
        sel = lane == farthest
        cx = jnp.sum(jnp.where(sel, x, 0.0), axis=1, keepdims=True)
        cy = jnp.sum(jnp.where(sel, y, 0.0), axis=1, keepdims=True)
        cz = jnp.sum(jnp.where(sel, z, 0.0), axis=1, keepdims=True)
        newxyz_ref[:, pl.ds(i, 1), :] = jnp.concatenate(
            [cx, cy, cz], axis=1)[:, None, :]
        dx = x - cx
        dy = y - cy
        dz = z - cz
        dist = dx * dx + dy * dy + dz * dz
        distance = jnp.minimum(distance, dist)
        farthest = jnp.argmax(distance, axis=1).astype(jnp.int32)[:, None]
        return distance, farthest

    dist0 = jnp.full((B, N), 1e10, jnp.float32)
    far0 = jnp.zeros((B, 1), jnp.int32)
    jax.lax.fori_loop(0, S, body, (dist0, far0))


def _run_fps(xyz):
    return pl.pallas_call(
        _fps_kernel,
        out_shape=jax.ShapeDtypeStruct((B, S, 3), jnp.float32),
    )(xyz)


def _fw_kernel(f_ref, w_ref, out_ref):
    out_ref[0] = jax.lax.dot_general(
        f_ref[0], w_ref[...], (((1,), (0,)), ((), ())),
        preferred_element_type=jnp.float32)


def _run_fw(feats, w1cat_t):
    c_out = w1cat_t.shape[1]
    return pl.pallas_call(
        _fw_kernel,
        grid=(B,),
        in_specs=[
            pl.BlockSpec((1, N, feats.shape[2]), lambda b: (b, 0, 0)),
            pl.BlockSpec((feats.shape[2], c_out), lambda b: (0, 0)),
        ],
        out_specs=pl.BlockSpec((1, N, c_out), lambda b: (b, 0, 0)),
        out_shape=jax.ShapeDtypeStruct((B, N, c_out), jnp.float32),
    )(feats, w1cat_t)


def _group_l1_kernel(K, r2, sblk, c1,
                     nxyz_ref, xyz_ref, fw_ref, wxyz_ref, b1_ref,
                     y1_ref, ssum_ref, ssq_ref):
    a = nxyz_ref[0]
    p = xyz_ref[0]
    ab = jax.lax.dot_general(
        a, p, (((1,), (1,)), ((), ())),
        preferred_element_type=jnp.float32)
    d2 = (-2.0 * ab + jnp.sum(a * a, axis=1, keepdims=True)
          + jnp.sum(p * p, axis=1)[None, :])
    maskf = (d2 <= r2).astype(jnp.float32)
    mrow = jax.lax.broadcasted_iota(jnp.int32, (N, N), 0)
    ncol = jax.lax.broadcasted_iota(jnp.int32, (N, N), 1)
    lt = (mrow <= ncol).astype(jnp.float32)
    rank_incl = jax.lax.dot_general(
        maskf, lt, (((1,), (0,)), ((), ())),
        preferred_element_type=jnp.float32)
    rank_i = rank_incl.astype(jnp.int32)
    rank = jnp.where(maskf > 0.0, rank_i - 1, -1)
    count = rank_i[:, N - 1:N]
    kio = jax.lax.broadcasted_iota(jnp.int32, (sblk, K, N), 1)
    ranke = rank[:, None, :]
    counte = count[:, :, None]
    g = jnp.logical_or(ranke == kio,
                       jnp.logical_and(kio >= counte, ranke == 0))
    gf = g.astype(jnp.float32).reshape(sblk * K, N)
    fw = fw_ref[0]
    fw_hi = fw.astype(jnp.bfloat16).astype(jnp.float32)
    fw_lo = fw - fw_hi
    dn = (((1,), (0,)), ((), ()))
    y = (jax.lax.dot_general(gf, fw_hi, dn,
                             preferred_element_type=jnp.float32)
         + jax.lax.dot_general(gf, fw_lo, dn,
                               preferred_element_type=jnp.float32))
    corr = jax.lax.dot_general(
        a, wxyz_ref[...], (((1,), (0,)), ((), ())),
        preferred_element_type=jnp.float32)
    y = (y.reshape(sblk, K, c1)
         + (b1_ref[...] - corr)[:, None, :]).reshape(sblk * K, c1)
    y1_ref[0] = y

    @pl.when(jnp.logical_and(pl.program_id(0) == 0, pl.program_id(1) == 0))
    def _():
        ssum_ref[...] = jnp.zeros_like(ssum_ref)
        ssq_ref[...] = jnp.zeros_like(ssq_ref)

    ssum_ref[...] += jnp.sum(y, axis=0, keepdims=True)
    ssq_ref[...] += jnp.sum(y * y, axis=0, keepdims=True)


def _run_group_l1(newxyz, xyz_t, fw, w1xyz_t, b1, K, radius, sblk):
    c1 = fw.shape[2]
    r2 = radius * radius
    kern = functools.partial(_group_l1_kernel, K, r2, sblk, c1)
    return pl.pallas_call(
        kern,
        grid=(B, S // sblk),
        in_specs=[
            pl.BlockSpec((1, sblk, 3), lambda b, sb: (b, sb, 0)),
            pl.BlockSpec((1, N, 3), lambda b, sb: (b, 0, 0)),
            pl.BlockSpec((1, N, c1), lambda b, sb: (b, 0, 0)),
            pl.BlockSpec((3, c1), lambda b, sb: (0, 0)),
            pl.BlockSpec((1, c1), lambda b, sb: (0, 0)),
        ],
        out_specs=(
            pl.BlockSpec((1, sblk * K, c1), lambda b, sb: (b, sb, 0)),
            pl.BlockSpec((1, c1), lambda b, sb: (0, 0)),
            pl.BlockSpec((1, c1), lambda b, sb: (0, 0)),
        ),
        out_shape=(
            jax.ShapeDtypeStruct((B, S * K, c1), jnp.float32),
            jax.ShapeDtypeStruct((1, c1), jnp.float32),
            jax.ShapeDtypeStruct((1, c1), jnp.float32),
        ),
    )(newxyz, xyz_t, fw, w1xyz_t, b1)


def _layer_kernel(yp_ref, sc_ref, sh_ref, w_ref, b_ref,
                  y_ref, ssum_ref, ssq_ref):
    h = jnp.maximum(yp_ref[...] * sc_ref[...] + sh_ref[...], 0.0)
    y = jax.lax.dot_general(
        h, w_ref[...], (((1,), (0,)), ((), ())),
        preferred_element_type=jnp.float32) + b_ref[...]
    y_ref[...] = y

    @pl.when(pl.program_id(0) == 0)
    def _():
        ssum_ref[...] = jnp.zeros_like(ssum_ref)
        ssq_ref[...] = jnp.zeros_like(ssq_ref)

    ssum_ref[...] += jnp.sum(y, axis=0, keepdims=True)
    ssq_ref[...] += jnp.sum(y * y, axis=0, keepdims=True)


def _run_layer(yp, scale, shift, w_t, b, rblk):
    m, cin = yp.shape
    cout = w_t.shape[1]
    return pl.pallas_call(
        _layer_kernel,
        grid=(m // rblk,),
        in_specs=[
            pl.BlockSpec((rblk, cin), lambda i: (i, 0)),
            pl.BlockSpec((1, cin), lambda i: (0, 0)),
            pl.BlockSpec((1, cin), lambda i: (0, 0)),
            pl.BlockSpec((cin, cout), lambda i: (0, 0)),
            pl.BlockSpec((1, cout), lambda i: (0, 0)),
        ],
        out_specs=(
            pl.BlockSpec((rblk, cout), lambda i: (i, 0)),
            pl.BlockSpec((1, cout), lambda i: (0, 0)),
            pl.BlockSpec((1, cout), lambda i: (0, 0)),
        ),
        out_shape=(
            jax.ShapeDtypeStruct((m, cout), jnp.float32),
            jax.ShapeDtypeStruct((1, cout), jnp.float32),
            jax.ShapeDtypeStruct((1, cout), jnp.float32),
        ),
    )(yp, scale, shift, w_t, b)


def _final_kernel(K, sblk, c3, yp_ref, sc_ref, sh_ref, out_ref):
    h = jnp.maximum(yp_ref[...] * sc_ref[...] + sh_ref[...], 0.0)
    out_ref[...] = jnp.max(h.reshape(sblk, K, c3), axis=1)


def _run_final(y3, scale, shift, K, sblk):
    m, c3 = y3.shape
    rows = m // K
    kern = functools.partial(_final_kernel, K, sblk, c3)
    return pl.pallas_call(
        kern,
        grid=(rows // sblk,),
        in_specs=[
            pl.BlockSpec((sblk * K, c3), lambda i: (i, 0)),
            pl.BlockSpec((1, c3), lambda i: (0, 0)),
            pl.BlockSpec((1, c3), lambda i: (0, 0)),
        ],
        out_specs=pl.BlockSpec((sblk, c3), lambda i: (i, 0)),
        out_shape=jax.ShapeDtypeStruct((rows, c3), jnp.float32),
    )(y3, scale, shift)


def _stats_to_affine(ssum, ssq, m_count, g, be):
    mean = ssum / m_count
    var = ssq / m_count - mean * mean
    inv = g * jax.lax.rsqrt(var + EPS)
    return inv, be - mean * inv


@jax.jit
def kernel(xyz, points, params):
    xyz_t = jnp.transpose(xyz, (0, 2, 1))
    points_t = jnp.transpose(points, (0, 2, 1))
    feats = jnp.concatenate([points_t, xyz_t], axis=2)

    newxyz = _run_fps(xyz)

    w1cat_t = jnp.transpose(
        jnp.concatenate([br[0]['W'] for br in params], axis=0))
    fw_all = _run_fw(feats, w1cat_t)

    outs = []
    c_off = 0
    sblk_l1 = (16, 16, 8)
    rblk = (4096, 4096, 4096)
    for i, (K, radius) in enumerate(zip(KS, RADII)):
        br = params[i]
        c1 = br[0]['W'].shape[0]
        fw = fw_all[:, :, c_off:c_off + c1]
        c_off += c1
        w1xyz_t = jnp.transpose(br[0]['W'][:, 320:])
        b1 = br[0]['b'][None, :]
        y1, s1, q1 = _run_group_l1(newxyz, xyz_t, fw, w1xyz_t, b1,
                                   K, radius, sblk_l1[i])
        m_count = float(B * S * K)
        y1 = y1.reshape(B * S * K, c1)
        sc1, sh1 = _stats_to_affine(s1, q1, m_count,
                                    br[0]['g'][None, :], br[0]['be'][None, :])
        y2, s2, q2 = _run_layer(y1, sc1, sh1,
                                jnp.transpose(br[1]['W']),
                                br[1]['b'][None, :], rblk[i])
        sc2, sh2 = _stats_to_affine(s2, q2, m_count,
                                    br[1]['g'][None, :], br[1]['be'][None, :])
        y3, s3, q3 = _run_layer(y2, sc2, sh2,
                                jnp.transpose(br[2]['W']),
                                br[2]['b'][None, :], rblk[i])
        sc3, sh3 = _stats_to_affine(s3, q3, m_count,
                                    br[2]['g'][None, :], br[2]['be'][None, :])
        pooled = _run_final(y3, sc3, sh3, K, 16)
        c3 = pooled.shape[1]
        outs.append(jnp.transpose(pooled.reshape(B, S, c3), (0, 2, 1)))

    new_xyz_out = jnp.transpose(newxyz, (0, 2, 1))
    return (new_xyz_out, jnp.concatenate(outs, axis=1))


# speedup vs baseline: 9.4254x; 1.1256x over previous
import functools

import jax
import jax.numpy as jnp
from jax.experimental import pallas as pl

B = 8
N = 512
S = 128
EPS = 1e-5
RADII = (0.2, 0.4, 0.8)
KS = (32, 64, 128)


def _fps_kernel(xyz_ref, newxyz_ref):
    x = xyz_ref[:, 0, :]
    y = xyz_ref[:, 1, :]
    z = xyz_ref[:, 2, :]
    lane = jax.lax.broadcasted_iota(jnp.int32, (B, N), 1)

    def body(i, state):
        distance, farthest = state
        sel = lane == farthest
        cx = jnp.sum(jnp.where(sel, x, 0.0), axis=1, keepdims=True)
        cy = jnp.sum(jnp.where(sel, y, 0.0), axis=1, keepdims=True)
        cz = jnp.sum(jnp.where(sel, z, 0.0), axis=1, keepdims=True)
        newxyz_ref[:, pl.ds(i, 1), :] = jnp.concatenate(
            [cx, cy, cz], axis=1)[:, None, :]
        dx = x - cx
        dy = y - cy
        dz = z - cz
        dist = dx * dx + dy * dy + dz * dz
        distance = jnp.minimum(distance, dist)
        farthest = jnp.argmax(distance, axis=1).astype(jnp.int32)[:, None]
        return distance, farthest

    dist0 = jnp.full((B, N), 1e10, jnp.float32)
    far0 = jnp.zeros((B, 1), jnp.int32)
    jax.lax.fori_loop(0, S, body, (dist0, far0))


def _run_fps(xyz):
    return pl.pallas_call(
        _fps_kernel,
        out_shape=jax.ShapeDtypeStruct((B, S, 3), jnp.float32),
    )(xyz)


def _fw_kernel(f_ref, w_ref, hi_ref, lo_ref):
    fw = jax.lax.dot_general(
        f_ref[0], w_ref[...], (((1,), (0,)), ((), ())),
        preferred_element_type=jnp.float32)
    hi = fw.astype(jnp.bfloat16)
    hi_ref[0] = hi
    lo_ref[0] = (fw - hi.astype(jnp.float32)).astype(jnp.bfloat16)


def _run_fw(feats, w1cat_t):
    c_out = w1cat_t.shape[1]
    return pl.pallas_call(
        _fw_kernel,
        grid=(B,),
        in_specs=[
            pl.BlockSpec((1, N, feats.shape[2]), lambda b: (b, 0, 0)),
            pl.BlockSpec((feats.shape[2], c_out), lambda b: (0, 0)),
        ],
        out_specs=(
            pl.BlockSpec((1, N, c_out), lambda b: (b, 0, 0)),
            pl.BlockSpec((1, N, c_out), lambda b: (b, 0, 0)),
        ),
        out_shape=(
            jax.ShapeDtypeStruct((B, N, c_out), jnp.bfloat16),
            jax.ShapeDtypeStruct((B, N, c_out), jnp.bfloat16),
        ),
    )(feats, w1cat_t)


def _group_l1_kernel(K, r2, sblk, c1,
                     nxyz_ref, xyz_ref, hi_ref, lo_ref, lt_ref,
                     wxyz_ref, b1_ref,
                     y1_ref, ssum_ref, ssq_ref):
    a = nxyz_ref[0]
    p = xyz_ref[0]
    ab = jax.lax.dot_general(
        a, p, (((1,), (1,)), ((), ())),
        preferred_element_type=jnp.float32)
    d2 = (-2.0 * ab + jnp.sum(a * a, axis=1, keepdims=True)
          + jnp.sum(p * p, axis=1)[None, :])
    mask = d2 <= r2
    rank_incl = jax.lax.dot_general(
        mask.astype(jnp.bfloat16), lt_ref[...], (((1,), (0,)), ((), ())),
        preferred_element_type=jnp.float32)
    rank_i = rank_incl.astype(jnp.int32)
    rank = jnp.where(mask, rank_i - 1, -1)
    count = rank_i[:, N - 1:N]
    kio = jax.lax.broadcasted_iota(jnp.int32, (sblk, K, N), 1)
    gb = (rank[:, None, :] == kio).astype(jnp.bfloat16).reshape(sblk * K, N)
    dn = (((1,), (0,)), ((), ()))
    y = (jax.lax.dot_general(gb, hi_ref[0], dn,
                             preferred_element_type=jnp.float32)
         + jax.lax.dot_general(gb, lo_ref[0], dn,
                               preferred_element_type=jnp.float32))
    corr = jax.lax.dot_general(
        a, wxyz_ref[...], (((1,), (0,)), ((), ())),
        preferred_element_type=jnp.float32)
    y = y.reshape(sblk, K, c1) + (b1_ref[...] - corr)[:, None, :]
    kio2 = jax.lax.broadcasted_iota(jnp.int32, (sblk, K, c1), 1)
    y = jnp.where(kio2 >= count[:, :, None], y[:, 0:1, :], y)
    y = y.reshape(sblk * K, c1)
    y1_ref[0] = y

    @pl.when(jnp.logical_and(pl.program_id(0) == 0, pl.program_id(1) == 0))
    def _():
        ssum_ref[...] = jnp.zeros_like(ssum_ref)
        ssq_ref[...] = jnp.zeros_like(ssq_ref)

    ssum_ref[...] += jnp.sum(y, axis=0, keepdims=True)
    ssq_ref[...] += jnp.sum(y * y, axis=0, keepdims=True)


def _run_group_l1(newxyz, xyz_t, fw_hi, fw_lo, lt, w1xyz_t, b1,
                  K, radius, sblk):
    c1 = fw_hi.shape[2]
    r2 = radius * radius
    kern = functools.partial(_group_l1_kernel, K, r2, sblk, c1)
    return pl.pallas_call(
        kern,
        grid=(B, S // sblk),
        in_specs=[
            pl.BlockSpec((1, sblk, 3), lambda b, sb: (b, sb, 0)),
            pl.BlockSpec((1, N, 3), lambda b, sb: (b, 0, 0)),
            pl.BlockSpec((1, N, c1), lambda b, sb: (b, 0, 0)),
            pl.BlockSpec((1, N, c1), lambda b, sb: (b, 0, 0)),
            pl.BlockSpec((N, N), lambda b, sb: (0, 0)),
            pl.BlockSpec((3, c1), lambda b, sb: (0, 0)),
            pl.BlockSpec((1, c1), lambda b, sb: (0, 0)),
        ],
        out_specs=(
            pl.BlockSpec((1, sblk * K, c1), lambda b, sb: (b, sb, 0)),
            pl.BlockSpec((1, c1), lambda b, sb: (0, 0)),
            pl.BlockSpec((1, c1), lambda b, sb: (0, 0)),
        ),
        out_shape=(
            jax.ShapeDtypeStruct((B, S * K, c1), jnp.float32),
            jax.ShapeDtypeStruct((1, c1), jnp.float32),
            jax.ShapeDtypeStruct((1, c1), jnp.float32),
        ),
    )(newxyz, xyz_t, fw_hi, fw_lo, lt, w1xyz_t, b1)


def _layer_kernel(yp_ref, sc_ref, sh_ref, w_ref, b_ref,
                  y_ref, ssum_ref, ssq_ref):
    h = jnp.maximum(yp_ref[...] * sc_ref[...] + sh_ref[...], 0.0)
    y = jax.lax.dot_general(
        h, w_ref[...], (((1,), (0,)), ((), ())),
        preferred_element_type=jnp.float32) + b_ref[...]
    y_ref[...] = y

    @pl.when(pl.program_id(0) == 0)
    def _():
        ssum_ref[...] = jnp.zeros_like(ssum_ref)
        ssq_ref[...] = jnp.zeros_like(ssq_ref)

    ssum_ref[...] += jnp.sum(y, axis=0, keepdims=True)
    ssq_ref[...] += jnp.sum(y * y, axis=0, keepdims=True)


def _run_layer(yp, scale, shift, w_t, b, rblk):
    m, cin = yp.shape
    cout = w_t.shape[1]
    return pl.pallas_call(
        _layer_kernel,
        grid=(m // rblk,),
        in_specs=[
            pl.BlockSpec((rblk, cin), lambda i: (i, 0)),
            pl.BlockSpec((1, cin), lambda i: (0, 0)),
            pl.BlockSpec((1, cin), lambda i: (0, 0)),
            pl.BlockSpec((cin, cout), lambda i: (0, 0)),
            pl.BlockSpec((1, cout), lambda i: (0, 0)),
        ],
        out_specs=(
            pl.BlockSpec((rblk, cout), lambda i: (i, 0)),
            pl.BlockSpec((1, cout), lambda i: (0, 0)),
            pl.BlockSpec((1, cout), lambda i: (0, 0)),
        ),
        out_shape=(
            jax.ShapeDtypeStruct((m, cout), jnp.float32),
            jax.ShapeDtypeStruct((1, cout), jnp.float32),
            jax.ShapeDtypeStruct((1, cout), jnp.float32),
        ),
    )(yp, scale, shift, w_t, b)


def _final_kernel(K, sblk, c3, yp_ref, sc_ref, sh_ref, out_ref):
    h = jnp.maximum(yp_ref[...] * sc_ref[...] + sh_ref[...], 0.0)
    out_ref[...] = jnp.max(h.reshape(sblk, K, c3), axis=1)


def _run_final(y3, scale, shift, K, sblk):
    m, c3 = y3.shape
    rows = m // K
    kern = functools.partial(_final_kernel, K, sblk, c3)
    return pl.pallas_call(
        kern,
        grid=(rows // sblk,),
        in_specs=[
            pl.BlockSpec((sblk * K, c3), lambda i: (i, 0)),
            pl.BlockSpec((1, c3), lambda i: (0, 0)),
            pl.BlockSpec((1, c3), lambda i: (0, 0)),
        ],
        out_specs=pl.BlockSpec((sblk, c3), lambda i: (i, 0)),
        out_shape=jax.ShapeDtypeStruct((rows, c3), jnp.float32),
    )(y3, scale, shift)


def _stats_to_affine(ssum, ssq, m_count, g, be):
    mean = ssum / m_count
    var = ssq / m_count - mean * mean
    inv = g * jax.lax.rsqrt(var + EPS)
    return inv, be - mean * inv


@jax.jit
def kernel(xyz, points, params):
    xyz_t = jnp.transpose(xyz, (0, 2, 1))
    points_t = jnp.transpose(points, (0, 2, 1))
    feats = jnp.concatenate([points_t, xyz_t], axis=2)

    newxyz = _run_fps(xyz)

    w1cat_t = jnp.transpose(
        jnp.concatenate([br[0]['W'] for br in params], axis=0))
    fw_hi_all, fw_lo_all = _run_fw(feats, w1cat_t)
    lt = jnp.triu(jnp.ones((N, N), jnp.bfloat16))

    outs = []
    c_off = 0
    sblk_l1 = (16, 16, 8)
    rblk = (4096, 4096, 4096)
    for i, (K, radius) in enumerate(zip(KS, RADII)):
        br = params[i]
        c1 = br[0]['W'].shape[0]
        fw_hi = fw_hi_all[:, :, c_off:c_off + c1]
        fw_lo = fw_lo_all[:, :, c_off:c_off + c1]
        c_off += c1
        w1xyz_t = jnp.transpose(br[0]['W'][:, 320:])
        b1 = br[0]['b'][None, :]
        y1, s1, q1 = _run_group_l1(newxyz, xyz_t, fw_hi, fw_lo, lt,
                                   w1xyz_t, b1, K, radius, sblk_l1[i])
        m_count = float(B * S * K)
        y1 = y1.reshape(B * S * K, c1)
        sc1, sh1 = _stats_to_affine(s1, q1, m_count,
                                    br[0]['g'][None, :], br[0]['be'][None, :])
        y2, s2, q2 = _run_layer(y1, sc1, sh1,
                                jnp.transpose(br[1]['W']),
                                br[1]['b'][None, :], rblk[i])
        sc2, sh2 = _stats_to_affine(s2, q2, m_count,
                                    br[1]['g'][None, :], br[1]['be'][None, :])
        y3, s3, q3 = _run_layer(y2, sc2, sh2,
                                jnp.transpose(br[2]['W']),
                                br[2]['b'][None, :], rblk[i])
        sc3, sh3 = _stats_to_affine(s3, q3, m_count,
                                    br[2]['g'][None, :], br[2]['be'][None, :])
        pooled = _run_final(y3, sc3, sh3, K, 16)
        c3 = pooled.shape[1]
        outs.append(jnp.transpose(pooled.reshape(B, S, c3), (0, 2, 1)))

    new_xyz_out = jnp.transpose(newxyz, (0, 2, 1))
    return (new_xyz_out, jnp.concatenate(outs, axis=1))


# speedup vs baseline: 10.9698x; 1.1638x over previous
import functools

import jax
import jax.numpy as jnp
from jax.experimental import pallas as pl

B = 8
N = 512
S = 128
EPS = 1e-5
RADII = (0.2, 0.4, 0.8)
KS = (32, 64, 128)


def _fps_kernel(xyz_ref, newxyz_ref):
    x = xyz_ref[:, 0, :]
    y = xyz_ref[:, 1, :]
    z = xyz_ref[:, 2, :]
    lane = jax.lax.broadcasted_iota(jnp.int32, (B, N), 1)

    def body(i, state):
        distance, farthest = state
        sel = lane == farthest
        cx = jnp.sum(jnp.where(sel, x, 0.0), axis=1, keepdims=True)
        cy = jnp.sum(jnp.where(sel, y, 0.0), axis=1, keepdims=True)
        cz = jnp.sum(jnp.where(sel, z, 0.0), axis=1, keepdims=True)
        newxyz_ref[:, pl.ds(i, 1), :] = jnp.concatenate(
            [cx, cy, cz], axis=1)[:, None, :]
        dx = x - cx
        dy = y - cy
        dz = z - cz
        dist = dx * dx + dy * dy + dz * dz
        distance = jnp.minimum(distance, dist)
        farthest = jnp.argmax(distance, axis=1).astype(jnp.int32)[:, None]
        return distance, farthest

    dist0 = jnp.full((B, N), 1e10, jnp.float32)
    far0 = jnp.zeros((B, 1), jnp.int32)
    jax.lax.fori_loop(0, S, body, (dist0, far0))


def _run_fps(xyz):
    return pl.pallas_call(
        _fps_kernel,
        out_shape=jax.ShapeDtypeStruct((B, S, 3), jnp.float32),
    )(xyz)


def _fw_kernel(pts_ref, xyz_ref, wp_ref, wx_ref, hi_ref, lo_ref):
    dn0 = (((0,), (0,)), ((), ()))
    fw = (jax.lax.dot_general(pts_ref[0], wp_ref[...], dn0,
                              preferred_element_type=jnp.float32)
          + jax.lax.dot_general(xyz_ref[0], wx_ref[...], dn0,
                                preferred_element_type=jnp.float32))
    hi = fw.astype(jnp.bfloat16)
    hi_ref[0] = hi
    lo_ref[0] = (fw - hi.astype(jnp.float32)).astype(jnp.bfloat16)


def _run_fw(points, xyz, wp_t, wx_t):
    c_out = wp_t.shape[1]
    c_in = wp_t.shape[0]
    return pl.pallas_call(
        _fw_kernel,
        grid=(B,),
        in_specs=[
            pl.BlockSpec((1, c_in, N), lambda b: (b, 0, 0)),
            pl.BlockSpec((1, 3, N), lambda b: (b, 0, 0)),
            pl.BlockSpec((c_in, c_out), lambda b: (0, 0)),
            pl.BlockSpec((3, c_out), lambda b: (0, 0)),
        ],
        out_specs=(
            pl.BlockSpec((1, N, c_out), lambda b: (b, 0, 0)),
            pl.BlockSpec((1, N, c_out), lambda b: (b, 0, 0)),
        ),
        out_shape=(
            jax.ShapeDtypeStruct((B, N, c_out), jnp.bfloat16),
            jax.ShapeDtypeStruct((B, N, c_out), jnp.bfloat16),
        ),
    )(points, xyz, wp_t, wx_t)


def _group_l1_kernel(K, r2, sblk, c1,
                     nxyz_ref, xyz_ref, hi_ref, lo_ref, lt_ref,
                     wxyz_ref, b1_ref,
                     y1_ref, ssum_ref, ssq_ref):
    a = nxyz_ref[0]
    p = xyz_ref[0]
    ab = jax.lax.dot_general(
        a, p, (((1,), (0,)), ((), ())),
        preferred_element_type=jnp.float32)
    d2 = (-2.0 * ab + jnp.sum(a * a, axis=1, keepdims=True)
          + jnp.sum(p * p, axis=0)[None, :])
    mask = d2 <= r2
    rank_incl = jax.lax.dot_general(
        mask.astype(jnp.bfloat16), lt_ref[...], (((1,), (0,)), ((), ())),
        preferred_element_type=jnp.float32)
    rank_i = rank_incl.astype(jnp.int32)
    rank = jnp.where(mask, rank_i - 1, -1)
    count = rank_i[:, N - 1:N]
    kio = jax.lax.broadcasted_iota(jnp.int32, (sblk, K, N), 1)
    gb = (rank[:, None, :] == kio).astype(jnp.bfloat16).reshape(sblk * K, N)
    dn = (((1,), (0,)), ((), ()))
    y = (jax.lax.dot_general(gb, hi_ref[0][:, :c1], dn,
                             preferred_element_type=jnp.float32)
         + jax.lax.dot_general(gb, lo_ref[0][:, :c1], dn,
                               preferred_element_type=jnp.float32))
    corr = jax.lax.dot_general(
        a, wxyz_ref[...], (((1,), (0,)), ((), ())),
        preferred_element_type=jnp.float32)
    y = y.reshape(sblk, K, c1) + (b1_ref[...] - corr)[:, None, :]
    kio2 = jax.lax.broadcasted_iota(jnp.int32, (sblk, K, c1), 1)
    y = jnp.where(kio2 >= count[:, :, None], y[:, 0:1, :], y)
    y = y.reshape(sblk * K, c1)
    y1_ref[0] = y

    @pl.when(jnp.logical_and(pl.program_id(0) == 0, pl.program_id(1) == 0))
    def _():
        ssum_ref[...] = jnp.zeros_like(ssum_ref)
        ssq_ref[...] = jnp.zeros_like(ssq_ref)

    ssum_ref[...] += jnp.sum(y, axis=0, keepdims=True)
    ssq_ref[...] += jnp.sum(y * y, axis=0, keepdims=True)


def _run_group_l1(newxyz, xyz, fw_hi_all, fw_lo_all, lt, w1xyz_t, b1,
                  K, radius, sblk, c1, cblk):
    r2 = radius * radius
    kern = functools.partial(_group_l1_kernel, K, r2, sblk, c1)
    return pl.pallas_call(
        kern,
        grid=(B, S // sblk),
        in_specs=[
            pl.BlockSpec((1, sblk, 3), lambda b, sb: (b, sb, 0)),
            pl.BlockSpec((1, 3, N), lambda b, sb: (b, 0, 0)),
            pl.BlockSpec((1, N, 128), lambda b, sb: (b, 0, cblk)),
            pl.BlockSpec((1, N, 128), lambda b, sb: (b, 0, cblk)),
            pl.BlockSpec((N, N), lambda b, sb: (0, 0)),
            pl.BlockSpec((3, c1), lambda b, sb: (0, 0)),
            pl.BlockSpec((1, c1), lambda b, sb: (0, 0)),
        ],
        out_specs=(
            pl.BlockSpec((1, sblk * K, c1), lambda b, sb: (b, sb, 0)),
            pl.BlockSpec((1, c1), lambda b, sb: (0, 0)),
            pl.BlockSpec((1, c1), lambda b, sb: (0, 0)),
        ),
        out_shape=(
            jax.ShapeDtypeStruct((B, S * K, c1), jnp.float32),
            jax.ShapeDtypeStruct((1, c1), jnp.float32),
            jax.ShapeDtypeStruct((1, c1), jnp.float32),
        ),
    )(newxyz, xyz, fw_hi_all, fw_lo_all, lt, w1xyz_t, b1)


def _layer_kernel(yp_ref, sc_ref, sh_ref, w_ref, b_ref,
                  y_ref, ssum_ref, ssq_ref):
    h = jnp.maximum(yp_ref[...] * sc_ref[...] + sh_ref[...], 0.0)
    y = jax.lax.dot_general(
        h, w_ref[...], (((1,), (0,)), ((), ())),
        preferred_element_type=jnp.float32) + b_ref[...]
    y_ref[...] = y

    @pl.when(pl.program_id(0) == 0)
    def _():
        ssum_ref[...] = jnp.zeros_like(ssum_ref)
        ssq_ref[...] = jnp.zeros_like(ssq_ref)

    ssum_ref[...] += jnp.sum(y, axis=0, keepdims=True)
    ssq_ref[...] += jnp.sum(y * y, axis=0, keepdims=True)


def _run_layer(yp, scale, shift, w_t, b, rblk):
    m, cin = yp.shape
    cout = w_t.shape[1]
    return pl.pallas_call(
        _layer_kernel,
        grid=(m // rblk,),
        in_specs=[
            pl.BlockSpec((rblk, cin), lambda i: (i, 0)),
            pl.BlockSpec((1, cin), lambda i: (0, 0)),
            pl.BlockSpec((1, cin), lambda i: (0, 0)),
            pl.BlockSpec((cin, cout), lambda i: (0, 0)),
            pl.BlockSpec((1, cout), lambda i: (0, 0)),
        ],
        out_specs=(
            pl.BlockSpec((rblk, cout), lambda i: (i, 0)),
            pl.BlockSpec((1, cout), lambda i: (0, 0)),
            pl.BlockSpec((1, cout), lambda i: (0, 0)),
        ),
        out_shape=(
            jax.ShapeDtypeStruct((m, cout), jnp.float32),
            jax.ShapeDtypeStruct((1, cout), jnp.float32),
            jax.ShapeDtypeStruct((1, cout), jnp.float32),
        ),
    )(yp, scale, shift, w_t, b)


def _final_kernel(K, sblk, c3, yp_ref, sc_ref, sh_ref, out_ref):
    h = jnp.maximum(yp_ref[...] * sc_ref[...] + sh_ref[...], 0.0)
    out_ref[0] = jnp.transpose(jnp.max(h.reshape(sblk, K, c3), axis=1))


def _run_final(y3, scale, shift, K):
    m, c3 = y3.shape
    kern = functools.partial(_final_kernel, K, S, c3)
    return pl.pallas_call(
        kern,
        grid=(B,),
        in_specs=[
            pl.BlockSpec((S * K, c3), lambda b: (b, 0)),
            pl.BlockSpec((1, c3), lambda b: (0, 0)),
            pl.BlockSpec((1, c3), lambda b: (0, 0)),
        ],
        out_specs=pl.BlockSpec((1, c3, S), lambda b: (b, 0, 0)),
        out_shape=jax.ShapeDtypeStruct((B, c3, S), jnp.float32),
    )(y3, scale, shift)


def _stats_to_affine(ssum, ssq, m_count, g, be):
    mean = ssum / m_count
    var = ssq / m_count - mean * mean
    inv = g * jax.lax.rsqrt(var + EPS)
    return inv, be - mean * inv


@jax.jit
def kernel(xyz, points, params):
    newxyz = _run_fps(xyz)

    worder = (1, 2, 0)
    cblks = {1: 0, 2: 1, 0: 2}
    wcat = jnp.concatenate(
        [params[j][0]['W'] for j in worder]
        + [jnp.zeros((64, 323), jnp.float32)], axis=0)
    wp_t = jnp.transpose(wcat[:, :320])
    wx_t = jnp.transpose(wcat[:, 320:])
    fw_hi_all, fw_lo_all = _run_fw(points, xyz, wp_t, wx_t)
    lt = jnp.triu(jnp.ones((N, N), jnp.bfloat16))

    outs = []
    sblk_l1 = (16, 16, 8)
    rblk = (4096, 4096, 4096)
    for i, (K, radius) in enumerate(zip(KS, RADII)):
        br = params[i]
        c1 = br[0]['W'].shape[0]
        w1xyz_t = jnp.transpose(br[0]['W'][:, 320:])
        b1 = br[0]['b'][None, :]
        y1, s1, q1 = _run_group_l1(newxyz, xyz, fw_hi_all, fw_lo_all, lt,
                                   w1xyz_t, b1, K, radius, sblk_l1[i],
                                   c1, cblks[i])
        m_count = float(B * S * K)
        y1 = y1.reshape(B * S * K, c1)
        sc1, sh1 = _stats_to_affine(s1, q1, m_count,
                                    br[0]['g'][None, :], br[0]['be'][None, :])
        y2, s2, q2 = _run_layer(y1, sc1, sh1,
                                jnp.transpose(br[1]['W']),
                                br[1]['b'][None, :], rblk[i])
        sc2, sh2 = _stats_to_affine(s2, q2, m_count,
                                    br[1]['g'][None, :], br[1]['be'][None, :])
        y3, s3, q3 = _run_layer(y2, sc2, sh2,
                                jnp.transpose(br[2]['W']),
                                br[2]['b'][None, :], rblk[i])
        sc3, sh3 = _stats_to_affine(s3, q3, m_count,
                                    br[2]['g'][None, :], br[2]['be'][None, :])
        outs.append(_run_final(y3, sc3, sh3, K))

    new_xyz_out = jnp.transpose(newxyz, (0, 2, 1))
    return (new_xyz_out, jnp.concatenate(outs, axis=1))


# speedup vs baseline: 12.4411x; 1.1341x over previous
import functools

import jax
import jax.numpy as jnp
from jax.experimental import pallas as pl

B = 8
N = 512
S = 128
EPS = 1e-5
RADII = (0.2, 0.4, 0.8)
KS = (32, 64, 128)


def _fps_kernel(xyz_ref, newxyz_ref):
    x = xyz_ref[:, 0, :]
    y = xyz_ref[:, 1, :]
    z = xyz_ref[:, 2, :]
    lane = jax.lax.broadcasted_iota(jnp.int32, (B, N), 1)

    def body(i, state):
        distance, farthest = state
        sel = lane == farthest
        cx = jnp.sum(jnp.where(sel, x, 0.0), axis=1, keepdims=True)
        cy = jnp.sum(jnp.where(sel, y, 0.0), axis=1, keepdims=True)
        cz = jnp.sum(jnp.where(sel, z, 0.0), axis=1, keepdims=True)
        newxyz_ref[:, pl.ds(i, 1), :] = jnp.concatenate(
            [cx, cy, cz], axis=1)[:, None, :]
        dx = x - cx
        dy = y - cy
        dz = z - cz
        dist = dx * dx + dy * dy + dz * dz
        distance = jnp.minimum(distance, dist)
        farthest = jnp.argmax(distance, axis=1).astype(jnp.int32)[:, None]
        return distance, farthest

    dist0 = jnp.full((B, N), 1e10, jnp.float32)
    far0 = jnp.zeros((B, 1), jnp.int32)
    jax.lax.fori_loop(0, S, body, (dist0, far0))


def _run_fps(xyz):
    return pl.pallas_call(
        _fps_kernel,
        out_shape=jax.ShapeDtypeStruct((B, S, 3), jnp.float32),
    )(xyz)


def _fw_kernel(pts_ref, xyz_ref, wp_ref, wx_ref, hi_ref, lo_ref):
    dn0 = (((0,), (0,)), ((), ()))
    fw = (jax.lax.dot_general(pts_ref[0], wp_ref[...], dn0,
                              preferred_element_type=jnp.float32)
          + jax.lax.dot_general(xyz_ref[0], wx_ref[...], dn0,
                                preferred_element_type=jnp.float32))
    hi = fw.astype(jnp.bfloat16)
    hi_ref[0] = hi
    lo_ref[0] = (fw - hi.astype(jnp.float32)).astype(jnp.bfloat16)


def _run_fw(points, xyz, wp_t, wx_t):
    c_out = wp_t.shape[1]
    c_in = wp_t.shape[0]
    return pl.pallas_call(
        _fw_kernel,
        grid=(B,),
        in_specs=[
            pl.BlockSpec((1, c_in, N), lambda b: (b, 0, 0)),
            pl.BlockSpec((1, 3, N), lambda b: (b, 0, 0)),
            pl.BlockSpec((c_in, c_out), lambda b: (0, 0)),
            pl.BlockSpec((3, c_out), lambda b: (0, 0)),
        ],
        out_specs=(
            pl.BlockSpec((1, N, c_out), lambda b: (b, 0, 0)),
            pl.BlockSpec((1, N, c_out), lambda b: (b, 0, 0)),
        ),
        out_shape=(
            jax.ShapeDtypeStruct((B, N, c_out), jnp.bfloat16),
            jax.ShapeDtypeStruct((B, N, c_out), jnp.bfloat16),
        ),
    )(points, xyz, wp_t, wx_t)


def _group_l1_kernel(K, r2, sblk, c1,
                     nxyz_ref, xyz_ref, hi_ref, lo_ref, lt_ref,
                     wxyz_ref, b1_ref,
                     y1_ref, ssum_ref, ssq_ref):
    a = nxyz_ref[0]
    p = xyz_ref[0]
    ab = jax.lax.dot_general(
        a, p, (((1,), (0,)), ((), ())),
        preferred_element_type=jnp.float32)
    d2 = (-2.0 * ab + jnp.sum(a * a, axis=1, keepdims=True)
          + jnp.sum(p * p, axis=0)[None, :])
    mask = d2 <= r2
    rank_incl = jax.lax.dot_general(
        mask.astype(jnp.bfloat16), lt_ref[...], (((1,), (0,)), ((), ())),
        preferred_element_type=jnp.float32)
    rank_i = rank_incl.astype(jnp.int32)
    rank = jnp.where(mask, rank_i - 1, -1)
    count = rank_i[:, N - 1:N]
    kio = jax.lax.broadcasted_iota(jnp.int32, (sblk, K, N), 1)
    gb = (rank[:, None, :] == kio).astype(jnp.bfloat16).reshape(sblk * K, N)
    dn = (((1,), (0,)), ((), ()))
    y = (jax.lax.dot_general(gb, hi_ref[0][:, :c1], dn,
                             preferred_element_type=jnp.float32)
         + jax.lax.dot_general(gb, lo_ref[0][:, :c1], dn,
                               preferred_element_type=jnp.float32))
    corr = jax.lax.dot_general(
        a, wxyz_ref[...], (((1,), (0,)), ((), ())),
        preferred_element_type=jnp.float32)
    y = y.reshape(sblk, K, c1) + (b1_ref[...] - corr)[:, None, :]
    kio2 = jax.lax.broadcasted_iota(jnp.int32, (sblk, K, c1), 1)
    y = jnp.where(kio2 >= count[:, :, None], y[:, 0:1, :], y)
    y = y.reshape(sblk * K, c1)
    y1_ref[0] = y.astype(jnp.bfloat16)

    @pl.when(jnp.logical_and(pl.program_id(0) == 0, pl.program_id(1) == 0))
    def _():
        ssum_ref[...] = jnp.zeros_like(ssum_ref)
        ssq_ref[...] = jnp.zeros_like(ssq_ref)

    ssum_ref[...] += jnp.sum(y, axis=0, keepdims=True)
    ssq_ref[...] += jnp.sum(y * y, axis=0, keepdims=True)


def _run_group_l1(newxyz, xyz, fw_hi_all, fw_lo_all, lt, w1xyz_t, b1,
                  K, radius, sblk, c1, cblk):
    r2 = radius * radius
    kern = functools.partial(_group_l1_kernel, K, r2, sblk, c1)
    return pl.pallas_call(
        kern,
        grid=(B, S // sblk),
        in_specs=[
            pl.BlockSpec((1, sblk, 3), lambda b, sb: (b, sb, 0)),
            pl.BlockSpec((1, 3, N), lambda b, sb: (b, 0, 0)),
            pl.BlockSpec((1, N, 128), lambda b, sb: (b, 0, cblk)),
            pl.BlockSpec((1, N, 128), lambda b, sb: (b, 0, cblk)),
            pl.BlockSpec((N, N), lambda b, sb: (0, 0)),
            pl.BlockSpec((3, c1), lambda b, sb: (0, 0)),
            pl.BlockSpec((1, c1), lambda b, sb: (0, 0)),
        ],
        out_specs=(
            pl.BlockSpec((1, sblk * K, c1), lambda b, sb: (b, sb, 0)),
            pl.BlockSpec((1, c1), lambda b, sb: (0, 0)),
            pl.BlockSpec((1, c1), lambda b, sb: (0, 0)),
        ),
        out_shape=(
            jax.ShapeDtypeStruct((B, S * K, c1), jnp.bfloat16),
            jax.ShapeDtypeStruct((1, c1), jnp.float32),
            jax.ShapeDtypeStruct((1, c1), jnp.float32),
        ),
    )(newxyz, xyz, fw_hi_all, fw_lo_all, lt, w1xyz_t, b1)


def _affine_from_stats(psum_ref, psq_ref, g_ref, be_ref, minv):
    mean = psum_ref[...] * minv
    var = psq_ref[...] * minv - mean * mean
    sc = g_ref[...] * jax.lax.rsqrt(var + EPS)
    return sc, be_ref[...] - mean * sc


def _layer_kernel(minv, yp_ref, psum_ref, psq_ref, g_ref, be_ref, w_ref, b_ref,
                  y_ref, ssum_ref, ssq_ref):
    sc, sh = _affine_from_stats(psum_ref, psq_ref, g_ref, be_ref, minv)
    h = jnp.maximum(yp_ref[...].astype(jnp.float32) * sc + sh, 0.0)
    y = jax.lax.dot_general(
        h, w_ref[...], (((1,), (0,)), ((), ())),
        preferred_element_type=jnp.float32) + b_ref[...]
    y_ref[...] = y.astype(jnp.bfloat16)

    @pl.when(pl.program_id(0) == 0)
    def _():
        ssum_ref[...] = jnp.zeros_like(ssum_ref)
        ssq_ref[...] = jnp.zeros_like(ssq_ref)

    ssum_ref[...] += jnp.sum(y, axis=0, keepdims=True)
    ssq_ref[...] += jnp.sum(y * y, axis=0, keepdims=True)


def _run_layer(yp, psum, psq, g, be, minv, w_t, b, rblk):
    m, cin = yp.shape
    cout = w_t.shape[1]
    kern = functools.partial(_layer_kernel, minv)
    return pl.pallas_call(
        kern,
        grid=(m // rblk,),
        in_specs=[
            pl.BlockSpec((rblk, cin), lambda i: (i, 0)),
            pl.BlockSpec((1, cin), lambda i: (0, 0)),
            pl.BlockSpec((1, cin), lambda i: (0, 0)),
            pl.BlockSpec((1, cin), lambda i: (0, 0)),
            pl.BlockSpec((1, cin), lambda i: (0, 0)),
            pl.BlockSpec((cin, cout), lambda i: (0, 0)),
            pl.BlockSpec((1, cout), lambda i: (0, 0)),
        ],
        out_specs=(
            pl.BlockSpec((rblk, cout), lambda i: (i, 0)),
            pl.BlockSpec((1, cout), lambda i: (0, 0)),
            pl.BlockSpec((1, cout), lambda i: (0, 0)),
        ),
        out_shape=(
            jax.ShapeDtypeStruct((m, cout), jnp.bfloat16),
            jax.ShapeDtypeStruct((1, cout), jnp.float32),
            jax.ShapeDtypeStruct((1, cout), jnp.float32),
        ),
    )(yp, psum, psq, g, be, w_t, b)


def _final_kernel(K, c3, minv, yp_ref, psum_ref, psq_ref, g_ref, be_ref,
                  out_ref):
    sc, sh = _affine_from_stats(psum_ref, psq_ref, g_ref, be_ref, minv)
    h = jnp.maximum(yp_ref[...].astype(jnp.float32) * sc + sh, 0.0)
    out_ref[0] = jnp.transpose(jnp.max(h.reshape(S, K, c3), axis=1))


def _run_final(y3, psum, psq, g, be, minv, K):
    m, c3 = y3.shape
    kern = functools.partial(_final_kernel, K, c3, minv)
    return pl.pallas_call(
        kern,
        grid=(B,),
        in_specs=[
            pl.BlockSpec((S * K, c3), lambda b: (b, 0)),
            pl.BlockSpec((1, c3), lambda b: (0, 0)),
            pl.BlockSpec((1, c3), lambda b: (0, 0)),
            pl.BlockSpec((1, c3), lambda b: (0, 0)),
            pl.BlockSpec((1, c3), lambda b: (0, 0)),
        ],
        out_specs=pl.BlockSpec((1, c3, S), lambda b: (b, 0, 0)),
        out_shape=jax.ShapeDtypeStruct((B, c3, S), jnp.float32),
    )(y3, psum, psq, g, be)


@jax.jit
def kernel(xyz, points, params):
    newxyz = _run_fps(xyz)

    worder = (1, 2, 0)
    cblks = {1: 0, 2: 1, 0: 2}
    wcat = jnp.concatenate(
        [params[j][0]['W'] for j in worder]
        + [jnp.zeros((64, 323), jnp.float32)], axis=0)
    wp_t = jnp.transpose(wcat[:, :320])
    wx_t = jnp.transpose(wcat[:, 320:])
    fw_hi_all, fw_lo_all = _run_fw(points, xyz, wp_t, wx_t)
    lt = jnp.triu(jnp.ones((N, N), jnp.bfloat16))

    outs = []
    sblk_l1 = (16, 16, 8)
    rblk = (4096, 4096, 4096)
    for i, (K, radius) in enumerate(zip(KS, RADII)):
        br = params[i]
        c1 = br[0]['W'].shape[0]
        w1xyz_t = jnp.transpose(br[0]['W'][:, 320:])
        b1 = br[0]['b'][None, :]
        y1, s1, q1 = _run_group_l1(newxyz, xyz, fw_hi_all, fw_lo_all, lt,
                                   w1xyz_t, b1, K, radius, sblk_l1[i],
                                   c1, cblks[i])
        minv = 1.0 / float(B * S * K)
        y1 = y1.reshape(B * S * K, c1)
        y2, s2, q2 = _run_layer(y1, s1, q1,
                                br[0]['g'][None, :], br[0]['be'][None, :],
                                minv, jnp.transpose(br[1]['W']),
                                br[1]['b'][None, :], rblk[i])
        y3, s3, q3 = _run_layer(y2, s2, q2,
                                br[1]['g'][None, :], br[1]['be'][None, :],
                                minv, jnp.transpose(br[2]['W']),
                                br[2]['b'][None, :], rblk[i])
        outs.append(_run_final(y3, s3, q3,
                               br[2]['g'][None, :], br[2]['be'][None, :],
                               minv, K))

    new_xyz_out = jnp.transpose(newxyz, (0, 2, 1))
    return (new_xyz_out, jnp.concatenate(outs, axis=1))


# speedup vs baseline: 14.5269x; 1.1677x over previous
import functools

import jax
import jax.numpy as jnp
from jax.experimental import pallas as pl

B = 8
N = 512
S = 128
EPS = 1e-5
RADII = (0.2, 0.4, 0.8)
KS = (32, 64, 128)


def _fps_kernel(xyz_ref, newxyz_ref):
    x = xyz_ref[:, 0, :]
    y = xyz_ref[:, 1, :]
    z = xyz_ref[:, 2, :]
    lane = jax.lax.broadcasted_iota(jnp.int32, (B, N), 1)

    def body(i, state):
        distance, farthest = state
        sel = lane == farthest
        cx = jnp.sum(jnp.where(sel, x, 0.0), axis=1, keepdims=True)
        cy = jnp.sum(jnp.where(sel, y, 0.0), axis=1, keepdims=True)
        cz = jnp.sum(jnp.where(sel, z, 0.0), axis=1, keepdims=True)
        newxyz_ref[:, pl.ds(i, 1), :] = jnp.concatenate(
            [cx, cy, cz], axis=1)[:, None, :]
        dx = x - cx
        dy = y - cy
        dz = z - cz
        dist = dx * dx + dy * dy + dz * dz
        distance = jnp.minimum(distance, dist)
        farthest = jnp.argmax(distance, axis=1).astype(jnp.int32)[:, None]
        return distance, farthest

    dist0 = jnp.full((B, N), 1e10, jnp.float32)
    far0 = jnp.zeros((B, 1), jnp.int32)
    jax.lax.fori_loop(0, S, body, (dist0, far0))


def _run_fps(xyz):
    return pl.pallas_call(
        _fps_kernel,
        out_shape=jax.ShapeDtypeStruct((B, S, 3), jnp.float32),
    )(xyz)


def _fw_kernel(pts_ref, xyz_ref, wp_ref, wx_ref, hi_ref, lo_ref):
    dn0 = (((0,), (0,)), ((), ()))
    fw = (jax.lax.dot_general(pts_ref[0], wp_ref[...], dn0,
                              preferred_element_type=jnp.float32)
          + jax.lax.dot_general(xyz_ref[0], wx_ref[...], dn0,
                                preferred_element_type=jnp.float32))
    hi = fw.astype(jnp.bfloat16)
    hi_ref[0] = hi
    lo_ref[0] = (fw - hi.astype(jnp.float32)).astype(jnp.bfloat16)


def _run_fw(points, xyz, wp_t, wx_t):
    c_out = wp_t.shape[1]
    c_in = wp_t.shape[0]
    return pl.pallas_call(
        _fw_kernel,
        grid=(B,),
        in_specs=[
            pl.BlockSpec((1, c_in, N), lambda b: (b, 0, 0)),
            pl.BlockSpec((1, 3, N), lambda b: (b, 0, 0)),
            pl.BlockSpec((c_in, c_out), lambda b: (0, 0)),
            pl.BlockSpec((3, c_out), lambda b: (0, 0)),
        ],
        out_specs=(
            pl.BlockSpec((1, N, c_out), lambda b: (b, 0, 0)),
            pl.BlockSpec((1, N, c_out), lambda b: (b, 0, 0)),
        ),
        out_shape=(
            jax.ShapeDtypeStruct((B, N, c_out), jnp.bfloat16),
            jax.ShapeDtypeStruct((B, N, c_out), jnp.bfloat16),
        ),
    )(points, xyz, wp_t, wx_t)


def _group_l1_kernel(K, r2, sblk, c1,
                     nxyz_ref, xyz_ref, hi_ref, lo_ref, lt_ref,
                     wxyz_ref, b1_ref,
                     y1_ref, ssum_ref, ssq_ref):
    a = nxyz_ref[0]
    p = xyz_ref[0]
    ab = jax.lax.dot_general(
        a, p, (((1,), (0,)), ((), ())),
        preferred_element_type=jnp.float32)
    d2 = (-2.0 * ab + jnp.sum(a * a, axis=1, keepdims=True)
          + jnp.sum(p * p, axis=0)[None, :])
    mask = d2 <= r2
    rank_incl = jax.lax.dot_general(
        mask.astype(jnp.bfloat16), lt_ref[...], (((1,), (0,)), ((), ())),
        preferred_element_type=jnp.float32)
    rank_i = rank_incl.astype(jnp.int32)
    rank = jnp.where(mask, rank_i - 1, -1)
    count = rank_i[:, N - 1:N]
    kio = jax.lax.broadcasted_iota(jnp.int32, (sblk, K, N), 1)
    gb = (rank[:, None, :] == kio).astype(jnp.bfloat16).reshape(sblk * K, N)
    dn = (((1,), (0,)), ((), ()))
    y = (jax.lax.dot_general(gb, hi_ref[0][:, :c1], dn,
                             preferred_element_type=jnp.float32)
         + jax.lax.dot_general(gb, lo_ref[0][:, :c1], dn,
                               preferred_element_type=jnp.float32))
    corr = jax.lax.dot_general(
        a, wxyz_ref[...], (((1,), (0,)), ((), ())),
        preferred_element_type=jnp.float32)
    y = y.reshape(sblk, K, c1) + (b1_ref[...] - corr)[:, None, :]
    kio2 = jax.lax.broadcasted_iota(jnp.int32, (sblk, K, c1), 1)
    y = jnp.where(kio2 >= count[:, :, None], y[:, 0:1, :], y)
    y = y.reshape(sblk * K, c1)
    y1_ref[0] = y.astype(jnp.bfloat16)

    @pl.when(jnp.logical_and(pl.program_id(0) == 0, pl.program_id(1) == 0))
    def _():
        ssum_ref[...] = jnp.zeros_like(ssum_ref)
        ssq_ref[...] = jnp.zeros_like(ssq_ref)

    ssum_ref[...] += jnp.sum(y, axis=0, keepdims=True)
    ssq_ref[...] += jnp.sum(y * y, axis=0, keepdims=True)


def _run_group_l1(newxyz, xyz, fw_hi_all, fw_lo_all, lt, w1xyz_t, b1,
                  K, radius, sblk, c1, cblk):
    r2 = radius * radius
    kern = functools.partial(_group_l1_kernel, K, r2, sblk, c1)
    return pl.pallas_call(
        kern,
        grid=(B, S // sblk),
        in_specs=[
            pl.BlockSpec((1, sblk, 3), lambda b, sb: (b, sb, 0)),
            pl.BlockSpec((1, 3, N), lambda b, sb: (b, 0, 0)),
            pl.BlockSpec((1, N, 128), lambda b, sb: (b, 0, cblk)),
            pl.BlockSpec((1, N, 128), lambda b, sb: (b, 0, cblk)),
            pl.BlockSpec((N, N), lambda b, sb: (0, 0)),
            pl.BlockSpec((3, c1), lambda b, sb: (0, 0)),
            pl.BlockSpec((1, c1), lambda b, sb: (0, 0)),
        ],
        out_specs=(
            pl.BlockSpec((1, sblk * K, c1), lambda b, sb: (b, sb, 0)),
            pl.BlockSpec((1, c1), lambda b, sb: (0, 0)),
            pl.BlockSpec((1, c1), lambda b, sb: (0, 0)),
        ),
        out_shape=(
            jax.ShapeDtypeStruct((B, S * K, c1), jnp.bfloat16),
            jax.ShapeDtypeStruct((1, c1), jnp.float32),
            jax.ShapeDtypeStruct((1, c1), jnp.float32),
        ),
    )(newxyz, xyz, fw_hi_all, fw_lo_all, lt, w1xyz_t, b1)


def _affine_from_stats(psum_ref, psq_ref, g_ref, be_ref, minv):
    mean = psum_ref[...] * minv
    var = psq_ref[...] * minv - mean * mean
    sc = g_ref[...] * jax.lax.rsqrt(var + EPS)
    return sc, be_ref[...] - mean * sc


def _layer_kernel(minv, yp_ref, psum_ref, psq_ref, g_ref, be_ref, w_ref, b_ref,
                  y_ref, ssum_ref, ssq_ref):
    sc, sh = _affine_from_stats(psum_ref, psq_ref, g_ref, be_ref, minv)
    h = jnp.maximum(yp_ref[...].astype(jnp.float32) * sc + sh, 0.0)
    y = jax.lax.dot_general(
        h, w_ref[...], (((1,), (0,)), ((), ())),
        preferred_element_type=jnp.float32) + b_ref[...]
    y_ref[...] = y.astype(jnp.bfloat16)

    @pl.when(pl.program_id(0) == 0)
    def _():
        ssum_ref[...] = jnp.zeros_like(ssum_ref)
        ssq_ref[...] = jnp.zeros_like(ssq_ref)

    ssum_ref[...] += jnp.sum(y, axis=0, keepdims=True)
    ssq_ref[...] += jnp.sum(y * y, axis=0, keepdims=True)


def _run_layer(yp, psum, psq, g, be, minv, w_t, b, rblk):
    m, cin = yp.shape
    cout = w_t.shape[1]
    kern = functools.partial(_layer_kernel, minv)
    return pl.pallas_call(
        kern,
        grid=(m // rblk,),
        in_specs=[
            pl.BlockSpec((rblk, cin), lambda i: (i, 0)),
            pl.BlockSpec((1, cin), lambda i: (0, 0)),
            pl.BlockSpec((1, cin), lambda i: (0, 0)),
            pl.BlockSpec((1, cin), lambda i: (0, 0)),
            pl.BlockSpec((1, cin), lambda i: (0, 0)),
            pl.BlockSpec((cin, cout), lambda i: (0, 0)),
            pl.BlockSpec((1, cout), lambda i: (0, 0)),
        ],
        out_specs=(
            pl.BlockSpec((rblk, cout), lambda i: (i, 0)),
            pl.BlockSpec((1, cout), lambda i: (0, 0)),
            pl.BlockSpec((1, cout), lambda i: (0, 0)),
        ),
        out_shape=(
            jax.ShapeDtypeStruct((m, cout), jnp.bfloat16),
            jax.ShapeDtypeStruct((1, cout), jnp.float32),
            jax.ShapeDtypeStruct((1, cout), jnp.float32),
        ),
    )(yp, psum, psq, g, be, w_t, b)


def _final_kernel(K, c3, minv, yp_ref, psum_ref, psq_ref, g_ref, be_ref,
                  out_ref):
    sc, sh = _affine_from_stats(psum_ref, psq_ref, g_ref, be_ref, minv)
    h = jnp.maximum(yp_ref[...].astype(jnp.float32) * sc + sh, 0.0)
    out_ref[0] = jnp.transpose(jnp.max(h.reshape(S, K, c3), axis=1))


def _run_final(y3, psum, psq, g, be, minv, K):
    m, c3 = y3.shape
    kern = functools.partial(_final_kernel, K, c3, minv)
    return pl.pallas_call(
        kern,
        grid=(B,),
        in_specs=[
            pl.BlockSpec((S * K, c3), lambda b: (b, 0)),
            pl.BlockSpec((1, c3), lambda b: (0, 0)),
            pl.BlockSpec((1, c3), lambda b: (0, 0)),
            pl.BlockSpec((1, c3), lambda b: (0, 0)),
            pl.BlockSpec((1, c3), lambda b: (0, 0)),
        ],
        out_specs=pl.BlockSpec((1, c3, S), lambda b: (b, 0, 0)),
        out_shape=jax.ShapeDtypeStruct((B, c3, S), jnp.float32),
    )(y3, psum, psq, g, be)


@jax.jit
def kernel(xyz, points, params):
    newxyz = _run_fps(xyz)

    worder = (1, 2, 0)
    cblks = {1: 0, 2: 1, 0: 2}
    wcat = jnp.concatenate(
        [params[j][0]['W'] for j in worder]
        + [jnp.zeros((64, 323), jnp.float32)], axis=0)
    wp_t = jnp.transpose(wcat[:, :320])
    wx_t = jnp.transpose(wcat[:, 320:])
    fw_hi_all, fw_lo_all = _run_fw(points, xyz, wp_t, wx_t)
    lt = jnp.triu(jnp.ones((N, N), jnp.bfloat16))

    outs = []
    sblk_l1 = (32, 32, 16)
    rblk = (8192, 8192, 8192)
    for i, (K, radius) in enumerate(zip(KS, RADII)):
        br = params[i]
        c1 = br[0]['W'].shape[0]
        w1xyz_t = jnp.transpose(br[0]['W'][:, 320:])
        b1 = br[0]['b'][None, :]
        y1, s1, q1 = _run_group_l1(newxyz, xyz, fw_hi_all, fw_lo_all, lt,
                                   w1xyz_t, b1, K, radius, sblk_l1[i],
                                   c1, cblks[i])
        minv = 1.0 / float(B * S * K)
        y1 = y1.reshape(B * S * K, c1)
        y2, s2, q2 = _run_layer(y1, s1, q1,
                                br[0]['g'][None, :], br[0]['be'][None, :],
                                minv, jnp.transpose(br[1]['W']),
                                br[1]['b'][None, :], rblk[i])
        y3, s3, q3 = _run_layer(y2, s2, q2,
                                br[1]['g'][None, :], br[1]['be'][None, :],
                                minv, jnp.transpose(br[2]['W']),
                                br[2]['b'][None, :], rblk[i])
        outs.append(_run_final(y3, s3, q3,
                               br[2]['g'][None, :], br[2]['be'][None, :],
                               minv, K))

    new_xyz_out = jnp.transpose(newxyz, (0, 2, 1))
    return (new_xyz_out, jnp.concatenate(outs, axis=1))


# speedup vs baseline: 15.1939x; 1.0459x over previous
import functools

import jax
import jax.numpy as jnp
from jax.experimental import pallas as pl

B = 8
N = 512
S = 128
EPS = 1e-5
RADII = (0.2, 0.4, 0.8)
KS = (32, 64, 128)


def _fps_kernel(xyz_ref, newxyz_ref):
    x = xyz_ref[:, 0, :]
    y = xyz_ref[:, 1, :]
    z = xyz_ref[:, 2, :]
    lane = jax.lax.broadcasted_iota(jnp.int32, (B, N), 1)

    def body(i, state):
        distance, farthest = state
        sel = lane == farthest
        cx = jnp.sum(jnp.where(sel, x, 0.0), axis=1, keepdims=True)
        cy = jnp.sum(jnp.where(sel, y, 0.0), axis=1, keepdims=True)
        cz = jnp.sum(jnp.where(sel, z, 0.0), axis=1, keepdims=True)
        newxyz_ref[:, pl.ds(i, 1), :] = jnp.concatenate(
            [cx, cy, cz], axis=1)[:, None, :]
        dx = x - cx
        dy = y - cy
        dz = z - cz
        dist = dx * dx + dy * dy + dz * dz
        distance = jnp.minimum(distance, dist)
        farthest = jnp.argmax(distance, axis=1).astype(jnp.int32)[:, None]
        return distance, farthest

    dist0 = jnp.full((B, N), 1e10, jnp.float32)
    far0 = jnp.zeros((B, 1), jnp.int32)
    jax.lax.fori_loop(0, S, body, (dist0, far0))


def _run_fps(xyz):
    return pl.pallas_call(
        _fps_kernel,
        out_shape=jax.ShapeDtypeStruct((B, S, 3), jnp.float32),
    )(xyz)


def _fw_kernel(pts_ref, xyz_ref, wp_ref, wx_ref, hi_ref, lo_ref):
    dn0 = (((0,), (0,)), ((), ()))
    fw = (jax.lax.dot_general(pts_ref[0], wp_ref[...], dn0,
                              preferred_element_type=jnp.float32)
          + jax.lax.dot_general(xyz_ref[0], wx_ref[...], dn0,
                                preferred_element_type=jnp.float32))
    hi = fw.astype(jnp.bfloat16)
    hi_ref[0] = hi
    lo_ref[0] = (fw - hi.astype(jnp.float32)).astype(jnp.bfloat16)


def _run_fw(points, xyz, wp_t, wx_t):
    c_out = wp_t.shape[1]
    c_in = wp_t.shape[0]
    return pl.pallas_call(
        _fw_kernel,
        grid=(B,),
        in_specs=[
            pl.BlockSpec((1, c_in, N), lambda b: (b, 0, 0)),
            pl.BlockSpec((1, 3, N), lambda b: (b, 0, 0)),
            pl.BlockSpec((c_in, c_out), lambda b: (0, 0)),
            pl.BlockSpec((3, c_out), lambda b: (0, 0)),
        ],
        out_specs=(
            pl.BlockSpec((1, N, c_out), lambda b: (b, 0, 0)),
            pl.BlockSpec((1, N, c_out), lambda b: (b, 0, 0)),
        ),
        out_shape=(
            jax.ShapeDtypeStruct((B, N, c_out), jnp.bfloat16),
            jax.ShapeDtypeStruct((B, N, c_out), jnp.bfloat16),
        ),
    )(points, xyz, wp_t, wx_t)


def _group_l1_kernel(K, r2, sblk, c1,
                     nxyz_ref, xyz_ref, hi_ref, lo_ref, lt_ref,
                     wxyz_ref, b1_ref,
                     y1_ref, ssum_ref, ssq_ref):
    a = nxyz_ref[0]
    p = xyz_ref[0]
    ab = jax.lax.dot_general(
        a, p, (((1,), (0,)), ((), ())),
        preferred_element_type=jnp.float32)
    d2 = (-2.0 * ab + jnp.sum(a * a, axis=1, keepdims=True)
          + jnp.sum(p * p, axis=0)[None, :])
    mask = d2 <= r2
    rank_incl = jax.lax.dot_general(
        mask.astype(jnp.bfloat16), lt_ref[...], (((1,), (0,)), ((), ())),
        preferred_element_type=jnp.float32)
    rank_i = rank_incl.astype(jnp.int32)
    rank = jnp.where(mask, rank_i - 1, -1)
    count = rank_i[:, N - 1:N]
    kio = jax.lax.broadcasted_iota(jnp.int32, (sblk, K, N), 1)
    gb = (rank[:, None, :] == kio).astype(jnp.bfloat16).reshape(sblk * K, N)
    dn = (((1,), (0,)), ((), ()))
    y = (jax.lax.dot_general(gb, hi_ref[0][:, :c1], dn,
                             preferred_element_type=jnp.float32)
         + jax.lax.dot_general(gb, lo_ref[0][:, :c1], dn,
                               preferred_element_type=jnp.float32))
    corr = jax.lax.dot_general(
        a, wxyz_ref[...], (((1,), (0,)), ((), ())),
        preferred_element_type=jnp.float32)
    y = y.reshape(sblk, K, c1) + (b1_ref[...] - corr)[:, None, :]
    kio2 = jax.lax.broadcasted_iota(jnp.int32, (sblk, K, c1), 1)
    y = jnp.where(kio2 >= count[:, :, None], y[:, 0:1, :], y)
    y = y.reshape(sblk * K, c1)
    y1_ref[0] = y.astype(jnp.bfloat16)

    @pl.when(jnp.logical_and(pl.program_id(0) == 0, pl.program_id(1) == 0))
    def _():
        ssum_ref[...] = jnp.zeros_like(ssum_ref)
        ssq_ref[...] = jnp.zeros_like(ssq_ref)

    ssum_ref[...] += jnp.sum(y, axis=0, keepdims=True)
    ssq_ref[...] += jnp.sum(y * y, axis=0, keepdims=True)


def _run_group_l1(newxyz, xyz, fw_hi_all, fw_lo_all, lt, w1xyz_t, b1,
                  K, radius, sblk, c1, cblk):
    r2 = radius * radius
    kern = functools.partial(_group_l1_kernel, K, r2, sblk, c1)
    return pl.pallas_call(
        kern,
        grid=(B, S // sblk),
        in_specs=[
            pl.BlockSpec((1, sblk, 3), lambda b, sb: (b, sb, 0)),
            pl.BlockSpec((1, 3, N), lambda b, sb: (b, 0, 0)),
            pl.BlockSpec((1, N, 128), lambda b, sb: (b, 0, cblk)),
            pl.BlockSpec((1, N, 128), lambda b, sb: (b, 0, cblk)),
            pl.BlockSpec((N, N), lambda b, sb: (0, 0)),
            pl.BlockSpec((3, c1), lambda b, sb: (0, 0)),
            pl.BlockSpec((1, c1), lambda b, sb: (0, 0)),
        ],
        out_specs=(
            pl.BlockSpec((1, sblk * K, c1), lambda b, sb: (b, sb, 0)),
            pl.BlockSpec((1, c1), lambda b, sb: (0, 0)),
            pl.BlockSpec((1, c1), lambda b, sb: (0, 0)),
        ),
        out_shape=(
            jax.ShapeDtypeStruct((B, S * K, c1), jnp.bfloat16),
            jax.ShapeDtypeStruct((1, c1), jnp.float32),
            jax.ShapeDtypeStruct((1, c1), jnp.float32),
        ),
    )(newxyz, xyz, fw_hi_all, fw_lo_all, lt, w1xyz_t, b1)


def _affine_from_stats(psum_ref, psq_ref, g_ref, be_ref, minv):
    mean = psum_ref[...] * minv
    var = psq_ref[...] * minv - mean * mean
    sc = g_ref[...] * jax.lax.rsqrt(var + EPS)
    return sc, be_ref[...] - mean * sc


def _layer_kernel(minv, yp_ref, psum_ref, psq_ref, g_ref, be_ref, w_ref, b_ref,
                  y_ref, ssum_ref, ssq_ref):
    sc, sh = _affine_from_stats(psum_ref, psq_ref, g_ref, be_ref, minv)
    h = jnp.maximum(yp_ref[...].astype(jnp.float32) * sc + sh, 0.0)
    y = jax.lax.dot_general(
        h, w_ref[...], (((1,), (0,)), ((), ())),
        preferred_element_type=jnp.float32) + b_ref[...]
    y_ref[...] = y.astype(jnp.bfloat16)

    @pl.when(pl.program_id(0) == 0)
    def _():
        ssum_ref[...] = jnp.zeros_like(ssum_ref)
        ssq_ref[...] = jnp.zeros_like(ssq_ref)

    ssum_ref[...] += jnp.sum(y, axis=0, keepdims=True)
    ssq_ref[...] += jnp.sum(y * y, axis=0, keepdims=True)


def _run_layer(yp, psum, psq, g, be, minv, w_t, b, rblk):
    m, cin = yp.shape
    cout = w_t.shape[1]
    kern = functools.partial(_layer_kernel, minv)
    return pl.pallas_call(
        kern,
        grid=(m // rblk,),
        in_specs=[
            pl.BlockSpec((rblk, cin), lambda i: (i, 0)),
            pl.BlockSpec((1, cin), lambda i: (0, 0)),
            pl.BlockSpec((1, cin), lambda i: (0, 0)),
            pl.BlockSpec((1, cin), lambda i: (0, 0)),
            pl.BlockSpec((1, cin), lambda i: (0, 0)),
            pl.BlockSpec((cin, cout), lambda i: (0, 0)),
            pl.BlockSpec((1, cout), lambda i: (0, 0)),
        ],
        out_specs=(
            pl.BlockSpec((rblk, cout), lambda i: (i, 0)),
            pl.BlockSpec((1, cout), lambda i: (0, 0)),
            pl.BlockSpec((1, cout), lambda i: (0, 0)),
        ),
        out_shape=(
            jax.ShapeDtypeStruct((m, cout), jnp.bfloat16),
            jax.ShapeDtypeStruct((1, cout), jnp.float32),
            jax.ShapeDtypeStruct((1, cout), jnp.float32),
        ),
    )(yp, psum, psq, g, be, w_t, b)


def _final_kernel(K, c3, minv, yp_ref, psum_ref, psq_ref, g_ref, be_ref,
                  out_ref):
    sc, sh = _affine_from_stats(psum_ref, psq_ref, g_ref, be_ref, minv)
    h = jnp.maximum(yp_ref[...].astype(jnp.float32) * sc + sh, 0.0)
    out_ref[0] = jnp.transpose(jnp.max(h.reshape(S, K, c3), axis=1))


def _run_final(y3, psum, psq, g, be, minv, K):
    m, c3 = y3.shape
    kern = functools.partial(_final_kernel, K, c3, minv)
    return pl.pallas_call(
        kern,
        grid=(B,),
        in_specs=[
            pl.BlockSpec((S * K, c3), lambda b: (b, 0)),
            pl.BlockSpec((1, c3), lambda b: (0, 0)),
            pl.BlockSpec((1, c3), lambda b: (0, 0)),
            pl.BlockSpec((1, c3), lambda b: (0, 0)),
            pl.BlockSpec((1, c3), lambda b: (0, 0)),
        ],
        out_specs=pl.BlockSpec((1, c3, S), lambda b: (b, 0, 0)),
        out_shape=jax.ShapeDtypeStruct((B, c3, S), jnp.float32),
    )(y3, psum, psq, g, be)


@jax.jit
def kernel(xyz, points, params):
    newxyz = _run_fps(xyz)

    worder = (1, 2, 0)
    cblks = {1: 0, 2: 1, 0: 2}
    wcat = jnp.concatenate(
        [params[j][0]['W'] for j in worder]
        + [jnp.zeros((64, 323), jnp.float32)], axis=0)
    wp_t = jnp.transpose(wcat[:, :320])
    wx_t = jnp.transpose(wcat[:, 320:])
    fw_hi_all, fw_lo_all = _run_fw(points, xyz, wp_t, wx_t)
    lt = jnp.triu(jnp.ones((N, N), jnp.bfloat16))

    outs = []
    sblk_l1 = (64, 32, 32)
    rblk = (16384, 16384, 16384)
    for i, (K, radius) in enumerate(zip(KS, RADII)):
        br = params[i]
        c1 = br[0]['W'].shape[0]
        w1xyz_t = jnp.transpose(br[0]['W'][:, 320:])
        b1 = br[0]['b'][None, :]
        y1, s1, q1 = _run_group_l1(newxyz, xyz, fw_hi_all, fw_lo_all, lt,
                                   w1xyz_t, b1, K, radius, sblk_l1[i],
                                   c1, cblks[i])
        minv = 1.0 / float(B * S * K)
        y1 = y1.reshape(B * S * K, c1)
        y2, s2, q2 = _run_layer(y1, s1, q1,
                                br[0]['g'][None, :], br[0]['be'][None, :],
                                minv, jnp.transpose(br[1]['W']),
                                br[1]['b'][None, :], rblk[i])
        y3, s3, q3 = _run_layer(y2, s2, q2,
                                br[1]['g'][None, :], br[1]['be'][None, :],
                                minv, jnp.transpose(br[2]['W']),
                                br[2]['b'][None, :], rblk[i])
        outs.append(_run_final(y3, s3, q3,
                               br[2]['g'][None, :], br[2]['be'][None, :],
                               minv, K))

    new_xyz_out = jnp.transpose(newxyz, (0, 2, 1))
    return (new_xyz_out, jnp.concatenate(outs, axis=1))
